# baseline (speedup 1.0000x reference)
import functools

import jax
import jax.numpy as jnp
from jax import lax
from jax.experimental import pallas as pl
from jax.experimental.pallas import tpu as pltpu
from jax.experimental.pallas import tpu_sc as plsc

NC = 2
NS = 16
NW = NC * NS
CH = 128
DW = 16


def _sc_mesh():
    return plsc.VectorSubcoreMesh(core_axis_name="c", subcore_axis_name="s")


def _row_split(n):
    rps = (n // NS) // 8 * 8
    return rps, n - NS * rps


@functools.lru_cache(maxsize=None)
def _make_deg(n, e):
    assert e % CH == 0
    nch = e // CH
    nfull = nch // NW
    rem = nch % NW
    rps, tail = _row_split(n)

    per_tile = nfull * CH

    @functools.partial(
        pl.kernel,
        out_type=jax.ShapeDtypeStruct((NC, n, DW), jnp.float32),
        mesh=_sc_mesh(),
        scratch_types=[
            pltpu.VMEM((2, per_tile), jnp.int32),
            pltpu.VMEM((CH,), jnp.int32),
            pltpu.VMEM((CH, DW), jnp.float32),
            pltpu.VMEM_SHARED((n, DW), jnp.float32),
            pltpu.SemaphoreType.DMA,
            pltpu.SemaphoreType.DMA,
        ],
    )
    def deg_kernel(ei_hbm, out_hbm, idxbuf, didx_s, ones, acc, sem, sem_idx):
        c = lax.axis_index("c")
        s = lax.axis_index("s")
        wid = s * NC + c
        ebase = pl.multiple_of(wid * per_tile, CH)
        pltpu.async_copy(ei_hbm.at[:, pl.ds(ebase, per_tile)], idxbuf, sem_idx)

        @pl.loop(0, CH)
        def _(j):
            ones[j, :] = jnp.ones((DW,), jnp.float32)

        r0 = pl.multiple_of(s * rps, 8)

        def init_range(base, count):
            off = 0
            while off < count:
                step = min(CH, count - off)
                pltpu.sync_copy(ones.at[pl.ds(0, step)],
                                acc.at[pl.ds(base + off, step)])
                off += step

        init_range(r0, rps)
        if tail:
            @pl.when(s == NS - 1)
            def _():
                init_range(NS * rps, tail)
        pltpu.make_async_copy(ei_hbm.at[:, pl.ds(ebase, per_tile)],
                              idxbuf, sem_idx).wait()
        plsc.subcore_barrier()

        def didx(i):
            return idxbuf.at[1, pl.ds(pl.multiple_of(i * CH, CH), CH)]

        @pl.loop(0, nfull)
        def _(i):
            pltpu.async_copy(ones, acc.at[didx(i)], sem, add=True)

        @pl.loop(0, nfull)
        def _(i):
            pltpu.make_async_copy(ones, acc.at[didx(i)], sem).wait()

        if rem:
            @pl.when(wid < rem)
            def _():
                off = pl.multiple_of((NW * nfull + wid) * CH, CH)
                pltpu.sync_copy(ei_hbm.at[1, pl.ds(off, CH)], didx_s)
                pltpu.sync_copy(ones, acc.at[didx_s], add=True)

        plsc.subcore_barrier()
        pltpu.sync_copy(acc.at[pl.ds(r0, rps)], out_hbm.at[c, pl.ds(r0, rps)])
        if tail:
            @pl.when(s == NS - 1)
            def _():
                pltpu.sync_copy(acc.at[pl.ds(NS * rps, tail)],
                                out_hbm.at[c, pl.ds(NS * rps, tail)])

    return deg_kernel


@functools.lru_cache(maxsize=None)
def _make_agg(n, h, e):
    assert e % CH == 0
    nch = e // CH
    nfull = nch // NW
    rem = nch % NW
    rps, tail = _row_split(n)

    per_tile = nfull * CH
    seg = nfull // 3
    assert seg >= 2 and seg % 2 == 0 and nfull == 3 * seg

    @functools.partial(
        pl.kernel,
        out_type=jax.ShapeDtypeStruct((NC, n, h), jnp.float32),
        mesh=_sc_mesh(),
        scratch_types=[
            pltpu.VMEM((2, seg * CH), jnp.int32),
            pltpu.VMEM((2, seg * CH), jnp.int32),
            pltpu.VMEM((CH, h), jnp.float32),
            pltpu.VMEM((CH, h), jnp.float32),
            pltpu.VMEM((CH,), jnp.int32),
            pltpu.VMEM((CH,), jnp.int32),
            pltpu.VMEM_SHARED((n, h), jnp.float32),
            pltpu.SemaphoreType.DMA,
            pltpu.SemaphoreType.DMA,
            pltpu.SemaphoreType.DMA,
        ],
    )
    def agg_kernel(y_hbm, ei_hbm, out_hbm, idxbuf_a, idxbuf_b, rows0, rows1,
                   sidx_s, didx_s, acc, sem0, sem1, sem_idx):
        c = lax.axis_index("c")
        s = lax.axis_index("s")
        wid = s * NC + c
        ebase = pl.multiple_of(wid * per_tile, CH)

        ibufs = (idxbuf_a, idxbuf_b)

        def stage_ref(t):
            return ei_hbm.at[:, pl.ds(ebase + t * seg * CH, seg * CH)]

        def stage(t, b):
            pltpu.async_copy(stage_ref(t), ibufs[b], sem_idx)

        def stage_wait(t, b):
            pltpu.make_async_copy(stage_ref(t), ibufs[b], sem_idx).wait()

        stage(0, 0)

        r0 = pl.multiple_of(s * rps, 8)

        @pl.when(c == 0)
        def _():
            pltpu.sync_copy(y_hbm.at[pl.ds(r0, rps)], acc.at[pl.ds(r0, rps)])
            if tail:
                @pl.when(s == NS - 1)
                def _():
                    pltpu.sync_copy(y_hbm.at[pl.ds(NS * rps, tail)],
                                    acc.at[pl.ds(NS * rps, tail)])

        @pl.when(c == 1)
        def _():
            @pl.loop(0, CH)
            def _(j):
                for q in range(h // 16):
                    rows0[j, pl.ds(16 * q, 16)] = jnp.zeros((16,), jnp.float32)
            nslab = rps // CH
            last = rps - nslab * CH
            @pl.loop(0, nslab)
            def _(j):
                pltpu.sync_copy(rows0, acc.at[pl.ds(r0 + j * CH, CH)])
            if last:
                pltpu.sync_copy(rows0.at[pl.ds(0, last)],
                                acc.at[pl.ds(r0 + nslab * CH, last)])
            if tail:
                @pl.when(s == NS - 1)
                def _():
                    pltpu.sync_copy(rows0.at[pl.ds(0, tail)],
                                    acc.at[pl.ds(NS * rps, tail)])

        stage_wait(0, 0)
        plsc.subcore_barrier()

        rows = (rows0, rows1)
        sems = (sem0, sem1)

        def run_pipe(ibuf):
            def sidx(i):
                return ibuf.at[0, pl.ds(pl.multiple_of(i * CH, CH), CH)]

            def didx(i):
                return ibuf.at[1, pl.ds(pl.multiple_of(i * CH, CH), CH)]

            def g_start(i, p):
                pltpu.async_copy(y_hbm.at[sidx(i)], rows[p], sems[p])

            def g_wait(i, p):
                pltpu.make_async_copy(y_hbm.at[sidx(i)], rows[p],
                                      sems[p]).wait()

            def scatter(i, p):
                pltpu.sync_copy(rows[p], acc.at[didx(i)], add=True)

            g_start(0, 0)

            @pl.loop(0, seg - 2, step=2)
            def _(i):
                g_start(i + 1, 1)
                g_wait(i, 0)
                scatter(i, 0)
                g_start(i + 2, 0)
                g_wait(i + 1, 1)
                scatter(i + 1, 1)

            g_start(seg - 1, 1)
            g_wait(seg - 2, 0)
            scatter(seg - 2, 0)
            g_wait(seg - 1, 1)
            scatter(seg - 1, 1)

        stage(1, 1)
        run_pipe(ibufs[0])
        stage_wait(1, 1)
        stage(2, 0)
        run_pipe(ibufs[1])
        stage_wait(2, 0)
        run_pipe(ibufs[0])

        if rem:
            @pl.when(wid < rem)
            def _():
                off = pl.multiple_of((NW * nfull + wid) * CH, CH)
                pltpu.sync_copy(ei_hbm.at[0, pl.ds(off, CH)], sidx_s)
                pltpu.sync_copy(ei_hbm.at[1, pl.ds(off, CH)], didx_s)
                pltpu.async_copy(y_hbm.at[sidx_s], rows0, sem0).wait()
                pltpu.sync_copy(rows0, acc.at[didx_s], add=True)

        plsc.subcore_barrier()
        pltpu.sync_copy(acc.at[pl.ds(r0, rps)], out_hbm.at[c, pl.ds(r0, rps)])
        if tail:
            @pl.when(s == NS - 1)
            def _():
                pltpu.sync_copy(acc.at[pl.ds(NS * rps, tail)],
                                out_hbm.at[c, pl.ds(NS * rps, tail)])

    return agg_kernel


def _dinv_from(dp_ref):
    deg = dp_ref[0, :, 0:1] + dp_ref[1, :, 0:1] - 1.0
    return lax.rsqrt(deg)


@functools.lru_cache(maxsize=None)
def _make_mm1(n, f, h, blk):
    def body(x_ref, w_ref, y_ref):
        y_ref[...] = jnp.dot(
            x_ref[...], w_ref[...], preferred_element_type=jnp.float32)

    return pl.pallas_call(
        body,
        grid=(n // blk,),
        in_specs=[
            pl.BlockSpec((blk, f), lambda i: (i, 0)),
            pl.BlockSpec((f, h), lambda i: (0, 0)),
        ],
        out_specs=pl.BlockSpec((blk, h), lambda i: (i, 0)),
        out_shape=jax.ShapeDtypeStruct((n, h), jnp.float32),
    )


@functools.lru_cache(maxsize=None)
def _make_scale1(n, h, blk):
    def body(xw_ref, dp_ref, y_ref):
        y_ref[...] = _dinv_from(dp_ref) * xw_ref[...]

    return pl.pallas_call(
        body,
        grid=(n // blk,),
        in_specs=[
            pl.BlockSpec((blk, h), lambda i: (i, 0)),
            pl.BlockSpec((NC, blk, DW), lambda i: (0, i, 0)),
        ],
        out_specs=pl.BlockSpec((blk, h), lambda i: (i, 0)),
        out_shape=jax.ShapeDtypeStruct((n, h), jnp.float32),
    )


@functools.lru_cache(maxsize=None)
def _make_mid(n, h, blk):
    def body(p_ref, dp_ref, w2_ref, b1_ref, y2_ref):
        dinv = _dinv_from(dp_ref)
        agg = p_ref[0] + p_ref[1]
        hact = jnp.maximum(dinv * agg + b1_ref[...], 0.0)
        y2_ref[...] = dinv * jnp.dot(
            hact, w2_ref[...], preferred_element_type=jnp.float32)

    return pl.pallas_call(
        body,
        grid=(n // blk,),
        in_specs=[
            pl.BlockSpec((NC, blk, h), lambda i: (0, i, 0)),
            pl.BlockSpec((NC, blk, DW), lambda i: (0, i, 0)),
            pl.BlockSpec((h, h), lambda i: (0, 0)),
            pl.BlockSpec((1, h), lambda i: (0, 0)),
        ],
        out_specs=pl.BlockSpec((blk, h), lambda i: (i, 0)),
        out_shape=jax.ShapeDtypeStruct((n, h), jnp.float32),
    )


@functools.lru_cache(maxsize=None)
def _make_emb_out(n, h, s_sub, blk):
    def body(p_ref, dp_ref, b2_ref, wl_ref, bl_ref, emb_ref, o_ref):
        dinv = _dinv_from(dp_ref)
        agg = p_ref[0] + p_ref[1]
        emb = dinv * agg + b2_ref[...]
        emb_ref[...] = emb
        z = emb.reshape(blk // s_sub, s_sub * h)
        o_ref[...] = lax.dot_general(
            z, wl_ref[...], (((1,), (1,)), ((), ())),
            preferred_element_type=jnp.float32) + bl_ref[...]

    return pl.pallas_call(
        body,
        grid=(n // blk,),
        in_specs=[
            pl.BlockSpec((NC, blk, h), lambda i: (0, i, 0)),
            pl.BlockSpec((NC, blk, DW), lambda i: (0, i, 0)),
            pl.BlockSpec((1, h), lambda i: (0, 0)),
            pl.BlockSpec((h, s_sub * h), lambda i: (0, 0)),
            pl.BlockSpec((1, h), lambda i: (0, 0)),
        ],
        out_specs=[
            pl.BlockSpec((blk, h), lambda i: (i, 0)),
            pl.BlockSpec((blk // s_sub, h), lambda i: (i, 0)),
        ],
        out_shape=[
            jax.ShapeDtypeStruct((n, h), jnp.float32),
            jax.ShapeDtypeStruct((n // s_sub, h), jnp.float32),
        ],
    )


def kernel(x, edge_index, W1, b1, W2, b2, Wl, bl):
    n, f = x.shape
    h = W1.shape[1]
    e = edge_index.shape[1]
    s_sub = Wl.shape[1] // h
    m = n // s_sub

    degp = _make_deg(n, e)(edge_index)
    xw1 = _make_mm1(n, f, h, 2000)(x, W1)
    y1 = _make_scale1(n, h, 2000)(xw1, degp)
    p1 = _make_agg(n, h, e)(y1, edge_index)
    y2 = _make_mid(n, h, 2000)(p1, degp, W2, b1.reshape(1, h))
    p2 = _make_agg(n, h, e)(y2, edge_index)
    emb, out = _make_emb_out(n, h, s_sub, 2000)(
        p2, degp, b2.reshape(1, h), Wl, bl.reshape(1, h))
    return (out, emb)


# speedup vs baseline: 1.0036x; 1.0036x over previous
import functools

import jax
import jax.numpy as jnp
from jax import lax
from jax.experimental import pallas as pl
from jax.experimental.pallas import tpu as pltpu
from jax.experimental.pallas import tpu_sc as plsc

NC = 2
NS = 16
NW = NC * NS
CH = 128
DW = 16


def _sc_mesh():
    return plsc.VectorSubcoreMesh(core_axis_name="c", subcore_axis_name="s")


def _row_split(n):
    rps = (n // NS) // 8 * 8
    return rps, n - NS * rps


@functools.lru_cache(maxsize=None)
def _make_deg(n, e):
    assert e % CH == 0
    nch = e // CH
    nfull = nch // NW
    rem = nch % NW
    rps, tail = _row_split(n)

    per_tile = nfull * CH

    @functools.partial(
        pl.kernel,
        out_type=jax.ShapeDtypeStruct((NC, n, DW), jnp.float32),
        mesh=_sc_mesh(),
        scratch_types=[
            pltpu.VMEM((2, per_tile), jnp.int32),
            pltpu.VMEM((CH,), jnp.int32),
            pltpu.VMEM((CH, DW), jnp.float32),
            pltpu.VMEM_SHARED((n, DW), jnp.float32),
            pltpu.SemaphoreType.DMA,
            pltpu.SemaphoreType.DMA,
        ],
    )
    def deg_kernel(ei_hbm, out_hbm, idxbuf, didx_s, ones, acc, sem, sem_idx):
        c = lax.axis_index("c")
        s = lax.axis_index("s")
        wid = s * NC + c
        ebase = pl.multiple_of(wid * per_tile, CH)
        pltpu.async_copy(ei_hbm.at[:, pl.ds(ebase, per_tile)], idxbuf, sem_idx)

        @pl.loop(0, CH)
        def _(j):
            ones[j, :] = jnp.ones((DW,), jnp.float32)

        r0 = pl.multiple_of(s * rps, 8)

        def init_range(base, count):
            off = 0
            while off < count:
                step = min(CH, count - off)
                pltpu.sync_copy(ones.at[pl.ds(0, step)],
                                acc.at[pl.ds(base + off, step)])
                off += step

        init_range(r0, rps)
        if tail:
            @pl.when(s == NS - 1)
            def _():
                init_range(NS * rps, tail)
        pltpu.make_async_copy(ei_hbm.at[:, pl.ds(ebase, per_tile)],
                              idxbuf, sem_idx).wait()
        plsc.subcore_barrier()

        def didx(i):
            return idxbuf.at[1, pl.ds(pl.multiple_of(i * CH, CH), CH)]

        @pl.loop(0, nfull)
        def _(i):
            pltpu.async_copy(ones, acc.at[didx(i)], sem, add=True)

        @pl.loop(0, nfull)
        def _(i):
            pltpu.make_async_copy(ones, acc.at[didx(i)], sem).wait()

        if rem:
            @pl.when(wid < rem)
            def _():
                off = pl.multiple_of((NW * nfull + wid) * CH, CH)
                pltpu.sync_copy(ei_hbm.at[1, pl.ds(off, CH)], didx_s)
                pltpu.sync_copy(ones, acc.at[didx_s], add=True)

        plsc.subcore_barrier()
        pltpu.sync_copy(acc.at[pl.ds(r0, rps)], out_hbm.at[c, pl.ds(r0, rps)])
        if tail:
            @pl.when(s == NS - 1)
            def _():
                pltpu.sync_copy(acc.at[pl.ds(NS * rps, tail)],
                                out_hbm.at[c, pl.ds(NS * rps, tail)])

    return deg_kernel


@functools.lru_cache(maxsize=None)
def _make_agg(n, h, e):
    assert e % CH == 0
    nch = e // CH
    nfull = nch // NW
    rem = nch % NW
    rps, tail = _row_split(n)

    assert nfull >= 4 and nfull % 2 == 0
    per_tile = nfull * CH
    cnt_a = (nfull // 2 + 1) // 2 * 2
    cnt_b = nfull - cnt_a
    assert cnt_b >= 2 and cnt_b % 2 == 0

    @functools.partial(
        pl.kernel,
        out_type=jax.ShapeDtypeStruct((NC, n, h), jnp.float32),
        mesh=_sc_mesh(),
        scratch_types=[
            pltpu.VMEM((2, cnt_a * CH), jnp.int32),
            pltpu.VMEM((CH, h), jnp.float32),
            pltpu.VMEM((CH, h), jnp.float32),
            pltpu.VMEM((CH,), jnp.int32),
            pltpu.VMEM((CH,), jnp.int32),
            pltpu.VMEM_SHARED((n, h), jnp.float32),
            pltpu.SemaphoreType.DMA,
            pltpu.SemaphoreType.DMA,
            pltpu.SemaphoreType.DMA,
        ],
    )
    def agg_kernel(y_hbm, ei_hbm, out_hbm, idxbuf, rows0, rows1,
                   sidx_s, didx_s, acc, sem0, sem1, sem_idx):
        c = lax.axis_index("c")
        s = lax.axis_index("s")
        wid = s * NC + c
        ebase = pl.multiple_of(wid * per_tile, CH)

        pltpu.async_copy(ei_hbm.at[:, pl.ds(ebase, cnt_a * CH)],
                         idxbuf, sem_idx)

        r0 = pl.multiple_of(s * rps, 8)

        @pl.when(c == 0)
        def _():
            pltpu.sync_copy(y_hbm.at[pl.ds(r0, rps)], acc.at[pl.ds(r0, rps)])
            if tail:
                @pl.when(s == NS - 1)
                def _():
                    pltpu.sync_copy(y_hbm.at[pl.ds(NS * rps, tail)],
                                    acc.at[pl.ds(NS * rps, tail)])

        @pl.when(c == 1)
        def _():
            @pl.loop(0, CH)
            def _(j):
                for q in range(h // 16):
                    rows0[j, pl.ds(16 * q, 16)] = jnp.zeros((16,), jnp.float32)
            nslab = rps // CH
            last = rps - nslab * CH
            @pl.loop(0, nslab)
            def _(j):
                pltpu.sync_copy(rows0, acc.at[pl.ds(r0 + j * CH, CH)])
            if last:
                pltpu.sync_copy(rows0.at[pl.ds(0, last)],
                                acc.at[pl.ds(r0 + nslab * CH, last)])
            if tail:
                @pl.when(s == NS - 1)
                def _():
                    pltpu.sync_copy(rows0.at[pl.ds(0, tail)],
                                    acc.at[pl.ds(NS * rps, tail)])

        pltpu.make_async_copy(ei_hbm.at[:, pl.ds(ebase, cnt_a * CH)],
                              idxbuf, sem_idx).wait()
        plsc.subcore_barrier()

        rows = (rows0, rows1)
        sems = (sem0, sem1)

        def sidx(i):
            return idxbuf.at[0, pl.ds(pl.multiple_of(i * CH, CH), CH)]

        def didx(i):
            return idxbuf.at[1, pl.ds(pl.multiple_of(i * CH, CH), CH)]

        def g_start(i, p):
            pltpu.async_copy(y_hbm.at[sidx(i)], rows[p], sems[p])

        def g_wait(i, p):
            pltpu.make_async_copy(y_hbm.at[sidx(i)], rows[p], sems[p]).wait()

        def scatter(i, p):
            pltpu.sync_copy(rows[p], acc.at[didx(i)], add=True)

        for c0, cnt in ((0, cnt_a), (cnt_a, cnt_b)):
            if c0:
                pltpu.sync_copy(ei_hbm.at[:, pl.ds(ebase + c0 * CH, cnt * CH)],
                                idxbuf.at[:, pl.ds(0, cnt * CH)])
            g_start(0, 0)

            @pl.loop(0, cnt - 2, step=2)
            def _(i):
                g_start(i + 1, 1)
                g_wait(i, 0)
                scatter(i, 0)
                g_start(i + 2, 0)
                g_wait(i + 1, 1)
                scatter(i + 1, 1)

            g_start(cnt - 1, 1)
            g_wait(cnt - 2, 0)
            scatter(cnt - 2, 0)
            g_wait(cnt - 1, 1)
            scatter(cnt - 1, 1)

        if rem:
            @pl.when(wid < rem)
            def _():
                off = pl.multiple_of((NW * nfull + wid) * CH, CH)
                pltpu.sync_copy(ei_hbm.at[0, pl.ds(off, CH)], sidx_s)
                pltpu.sync_copy(ei_hbm.at[1, pl.ds(off, CH)], didx_s)
                pltpu.async_copy(y_hbm.at[sidx_s], rows0, sem0).wait()
                pltpu.sync_copy(rows0, acc.at[didx_s], add=True)

        plsc.subcore_barrier()
        pltpu.sync_copy(acc.at[pl.ds(r0, rps)], out_hbm.at[c, pl.ds(r0, rps)])
        if tail:
            @pl.when(s == NS - 1)
            def _():
                pltpu.sync_copy(acc.at[pl.ds(NS * rps, tail)],
                                out_hbm.at[c, pl.ds(NS * rps, tail)])

    return agg_kernel


def _dinv_from(dp_ref):
    deg = dp_ref[0, :, 0:1] + dp_ref[1, :, 0:1] - 1.0
    return lax.rsqrt(deg)


@functools.lru_cache(maxsize=None)
def _make_mm1(n, f, h, blk):
    def body(x_ref, w_ref, y_ref):
        y_ref[...] = jnp.dot(
            x_ref[...], w_ref[...], preferred_element_type=jnp.float32)

    return pl.pallas_call(
        body,
        grid=(n // blk,),
        in_specs=[
            pl.BlockSpec((blk, f), lambda i: (i, 0)),
            pl.BlockSpec((f, h), lambda i: (0, 0)),
        ],
        out_specs=pl.BlockSpec((blk, h), lambda i: (i, 0)),
        out_shape=jax.ShapeDtypeStruct((n, h), jnp.float32),
    )


@functools.lru_cache(maxsize=None)
def _make_scale1(n, h, blk):
    def body(xw_ref, dp_ref, y_ref):
        y_ref[...] = _dinv_from(dp_ref) * xw_ref[...]

    return pl.pallas_call(
        body,
        grid=(n // blk,),
        in_specs=[
            pl.BlockSpec((blk, h), lambda i: (i, 0)),
            pl.BlockSpec((NC, blk, DW), lambda i: (0, i, 0)),
        ],
        out_specs=pl.BlockSpec((blk, h), lambda i: (i, 0)),
        out_shape=jax.ShapeDtypeStruct((n, h), jnp.float32),
    )


@functools.lru_cache(maxsize=None)
def _make_mid(n, h, blk):
    def body(p_ref, dp_ref, w2_ref, b1_ref, y2_ref):
        dinv = _dinv_from(dp_ref)
        agg = p_ref[0] + p_ref[1]
        hact = jnp.maximum(dinv * agg + b1_ref[...], 0.0)
        y2_ref[...] = dinv * jnp.dot(
            hact, w2_ref[...], preferred_element_type=jnp.float32)

    return pl.pallas_call(
        body,
        grid=(n // blk,),
        in_specs=[
            pl.BlockSpec((NC, blk, h), lambda i: (0, i, 0)),
            pl.BlockSpec((NC, blk, DW), lambda i: (0, i, 0)),
            pl.BlockSpec((h, h), lambda i: (0, 0)),
            pl.BlockSpec((1, h), lambda i: (0, 0)),
        ],
        out_specs=pl.BlockSpec((blk, h), lambda i: (i, 0)),
        out_shape=jax.ShapeDtypeStruct((n, h), jnp.float32),
    )


@functools.lru_cache(maxsize=None)
def _make_emb_out(n, h, s_sub, blk):
    def body(p_ref, dp_ref, b2_ref, wl_ref, bl_ref, emb_ref, o_ref):
        dinv = _dinv_from(dp_ref)
        agg = p_ref[0] + p_ref[1]
        emb = dinv * agg + b2_ref[...]
        emb_ref[...] = emb
        z = emb.reshape(blk // s_sub, s_sub * h)
        o_ref[...] = lax.dot_general(
            z, wl_ref[...], (((1,), (1,)), ((), ())),
            preferred_element_type=jnp.float32) + bl_ref[...]

    return pl.pallas_call(
        body,
        grid=(n // blk,),
        in_specs=[
            pl.BlockSpec((NC, blk, h), lambda i: (0, i, 0)),
            pl.BlockSpec((NC, blk, DW), lambda i: (0, i, 0)),
            pl.BlockSpec((1, h), lambda i: (0, 0)),
            pl.BlockSpec((h, s_sub * h), lambda i: (0, 0)),
            pl.BlockSpec((1, h), lambda i: (0, 0)),
        ],
        out_specs=[
            pl.BlockSpec((blk, h), lambda i: (i, 0)),
            pl.BlockSpec((blk // s_sub, h), lambda i: (i, 0)),
        ],
        out_shape=[
            jax.ShapeDtypeStruct((n, h), jnp.float32),
            jax.ShapeDtypeStruct((n // s_sub, h), jnp.float32),
        ],
    )


def kernel(x, edge_index, W1, b1, W2, b2, Wl, bl):
    n, f = x.shape
    h = W1.shape[1]
    e = edge_index.shape[1]
    s_sub = Wl.shape[1] // h
    m = n // s_sub

    degp = _make_deg(n, e)(edge_index)
    xw1 = _make_mm1(n, f, h, 2000)(x, W1)
    y1 = _make_scale1(n, h, 2000)(xw1, degp)
    p1 = _make_agg(n, h, e)(y1, edge_index)
    y2 = _make_mid(n, h, 2000)(p1, degp, W2, b1.reshape(1, h))
    p2 = _make_agg(n, h, e)(y2, edge_index)
    emb, out = _make_emb_out(n, h, s_sub, 2000)(
        p2, degp, b2.reshape(1, h), Wl, bl.reshape(1, h))
    return (out, emb)


# speedup vs baseline: 1.0072x; 1.0036x over previous
import functools

import jax
import jax.numpy as jnp
from jax import lax
from jax.experimental import pallas as pl
from jax.experimental.pallas import tpu as pltpu
from jax.experimental.pallas import tpu_sc as plsc

NC = 2
NS = 16
NW = NC * NS
CH = 128
DW = 16


def _sc_mesh():
    return plsc.VectorSubcoreMesh(core_axis_name="c", subcore_axis_name="s")


def _row_split(n):
    rps = (n // NS) // 8 * 8
    return rps, n - NS * rps


@functools.lru_cache(maxsize=None)
def _make_deg(n, e):
    assert e % CH == 0
    nch = e // CH
    nfull = nch // NW
    rem = nch % NW
    rps, tail = _row_split(n)

    per_tile = nfull * CH

    @functools.partial(
        pl.kernel,
        out_type=jax.ShapeDtypeStruct((NC, n, DW), jnp.float32),
        mesh=_sc_mesh(),
        scratch_types=[
            pltpu.VMEM((2, per_tile), jnp.int32),
            pltpu.VMEM((CH,), jnp.int32),
            pltpu.VMEM((CH, DW), jnp.float32),
            pltpu.VMEM_SHARED((n, DW), jnp.float32),
            pltpu.SemaphoreType.DMA,
            pltpu.SemaphoreType.DMA,
        ],
    )
    def deg_kernel(ei_hbm, out_hbm, idxbuf, didx_s, ones, acc, sem, sem_idx):
        c = lax.axis_index("c")
        s = lax.axis_index("s")
        wid = s * NC + c
        ebase = pl.multiple_of(wid * per_tile, CH)
        pltpu.async_copy(ei_hbm.at[:, pl.ds(ebase, per_tile)], idxbuf, sem_idx)

        @pl.loop(0, CH)
        def _(j):
            ones[j, :] = jnp.ones((DW,), jnp.float32)

        r0 = pl.multiple_of(s * rps, 8)

        def init_range(base, count):
            off = 0
            while off < count:
                step = min(CH, count - off)
                pltpu.sync_copy(ones.at[pl.ds(0, step)],
                                acc.at[pl.ds(base + off, step)])
                off += step

        init_range(r0, rps)
        if tail:
            @pl.when(s == NS - 1)
            def _():
                init_range(NS * rps, tail)
        pltpu.make_async_copy(ei_hbm.at[:, pl.ds(ebase, per_tile)],
                              idxbuf, sem_idx).wait()
        plsc.subcore_barrier()

        def didx(i):
            return idxbuf.at[1, pl.ds(pl.multiple_of(i * CH, CH), CH)]

        @pl.loop(0, nfull)
        def _(i):
            pltpu.async_copy(ones, acc.at[didx(i)], sem, add=True)

        @pl.loop(0, nfull)
        def _(i):
            pltpu.make_async_copy(ones, acc.at[didx(i)], sem).wait()

        if rem:
            @pl.when(wid < rem)
            def _():
                off = pl.multiple_of((NW * nfull + wid) * CH, CH)
                pltpu.sync_copy(ei_hbm.at[1, pl.ds(off, CH)], didx_s)
                pltpu.sync_copy(ones, acc.at[didx_s], add=True)

        plsc.subcore_barrier()
        pltpu.sync_copy(acc.at[pl.ds(r0, rps)], out_hbm.at[c, pl.ds(r0, rps)])
        if tail:
            @pl.when(s == NS - 1)
            def _():
                pltpu.sync_copy(acc.at[pl.ds(NS * rps, tail)],
                                out_hbm.at[c, pl.ds(NS * rps, tail)])

    return deg_kernel


@functools.lru_cache(maxsize=None)
def _make_agg(n, h, e):
    assert e % CH == 0
    nch = e // CH
    nfull = nch // NW
    rem = nch % NW
    rps, tail = _row_split(n)

    assert nfull >= 4 and nfull % 2 == 0
    per_tile = nfull * CH
    cnt_a = (nfull // 2 + 1) // 2 * 2
    cnt_b = nfull - cnt_a
    assert cnt_b >= 2 and cnt_b % 2 == 0

    @functools.partial(
        pl.kernel,
        out_type=jax.ShapeDtypeStruct((NC, n, h), jnp.float32),
        mesh=_sc_mesh(),
        scratch_types=[
            pltpu.VMEM((2, cnt_a * CH), jnp.int32),
            pltpu.VMEM((CH, h), jnp.float32),
            pltpu.VMEM((CH, h), jnp.float32),
            pltpu.VMEM((CH,), jnp.int32),
            pltpu.VMEM((CH,), jnp.int32),
            pltpu.VMEM_SHARED((n, h), jnp.float32),
            pltpu.SemaphoreType.DMA,
            pltpu.SemaphoreType.DMA,
            pltpu.SemaphoreType.DMA,
        ],
    )
    def agg_kernel(y_hbm, ei_hbm, out_hbm, idxbuf, rows0, rows1,
                   sidx_s, didx_s, acc, sem0, sem1, sem_idx):
        c = lax.axis_index("c")
        s = lax.axis_index("s")
        wid = s * NC + c
        ebase = pl.multiple_of(wid * per_tile, CH)

        pltpu.async_copy(ei_hbm.at[:, pl.ds(ebase, cnt_a * CH)],
                         idxbuf, sem_idx)

        r0 = pl.multiple_of(s * rps, 8)

        @pl.when(c == 0)
        def _():
            pltpu.sync_copy(y_hbm.at[pl.ds(r0, rps)], acc.at[pl.ds(r0, rps)])
            if tail:
                @pl.when(s == NS - 1)
                def _():
                    pltpu.sync_copy(y_hbm.at[pl.ds(NS * rps, tail)],
                                    acc.at[pl.ds(NS * rps, tail)])

        @pl.when(c == 1)
        def _():
            @pl.loop(0, CH)
            def _(j):
                for q in range(h // 16):
                    rows0[j, pl.ds(16 * q, 16)] = jnp.zeros((16,), jnp.float32)
            nslab = rps // CH
            last = rps - nslab * CH
            @pl.loop(0, nslab)
            def _(j):
                pltpu.sync_copy(rows0, acc.at[pl.ds(r0 + j * CH, CH)])
            if last:
                pltpu.sync_copy(rows0.at[pl.ds(0, last)],
                                acc.at[pl.ds(r0 + nslab * CH, last)])
            if tail:
                @pl.when(s == NS - 1)
                def _():
                    pltpu.sync_copy(rows0.at[pl.ds(0, tail)],
                                    acc.at[pl.ds(NS * rps, tail)])

        pltpu.make_async_copy(ei_hbm.at[:, pl.ds(ebase, cnt_a * CH)],
                              idxbuf, sem_idx).wait()
        plsc.subcore_barrier()

        rows = (rows0, rows1)
        sems = (sem0, sem1)

        def sidx(i):
            return idxbuf.at[0, pl.ds(pl.multiple_of(i * CH, CH), CH)]

        def didx(i):
            return idxbuf.at[1, pl.ds(pl.multiple_of(i * CH, CH), CH)]

        def g_start(i, p):
            pltpu.async_copy(y_hbm.at[sidx(i)], rows[p], sems[p])

        def g_wait(i, p):
            pltpu.make_async_copy(y_hbm.at[sidx(i)], rows[p], sems[p]).wait()

        def scatter(i, p):
            pltpu.sync_copy(rows[p], acc.at[didx(i)], add=True)

        for c0, cnt in ((0, cnt_a), (cnt_a, cnt_b)):
            if c0:
                pltpu.sync_copy(ei_hbm.at[:, pl.ds(ebase + c0 * CH, cnt * CH)],
                                idxbuf.at[:, pl.ds(0, cnt * CH)])
            g_start(0, 0)

            @pl.loop(0, cnt - 2, step=2)
            def _(i):
                g_start(i + 1, 1)
                g_wait(i, 0)
                scatter(i, 0)
                g_start(i + 2, 0)
                g_wait(i + 1, 1)
                scatter(i + 1, 1)

            g_start(cnt - 1, 1)
            g_wait(cnt - 2, 0)
            scatter(cnt - 2, 0)
            g_wait(cnt - 1, 1)
            scatter(cnt - 1, 1)

        if rem:
            @pl.when(wid < rem)
            def _():
                off = pl.multiple_of((NW * nfull + wid) * CH, CH)
                pltpu.sync_copy(ei_hbm.at[0, pl.ds(off, CH)], sidx_s)
                pltpu.sync_copy(ei_hbm.at[1, pl.ds(off, CH)], didx_s)
                pltpu.async_copy(y_hbm.at[sidx_s], rows0, sem0).wait()
                pltpu.sync_copy(rows0, acc.at[didx_s], add=True)

        plsc.subcore_barrier()
        pltpu.sync_copy(acc.at[pl.ds(r0, rps)], out_hbm.at[c, pl.ds(r0, rps)])
        if tail:
            @pl.when(s == NS - 1)
            def _():
                pltpu.sync_copy(acc.at[pl.ds(NS * rps, tail)],
                                out_hbm.at[c, pl.ds(NS * rps, tail)])

    return agg_kernel


def _dinv_from(dp_ref):
    deg = dp_ref[0, :, 0:1] + dp_ref[1, :, 0:1] - 1.0
    return lax.rsqrt(deg)


@functools.lru_cache(maxsize=None)
def _make_mm1(n, f, h, blk):
    def body(x_ref, w_ref, y_ref):
        y_ref[...] = jnp.dot(
            x_ref[...], w_ref[...], preferred_element_type=jnp.float32)

    return pl.pallas_call(
        body,
        grid=(n // blk,),
        in_specs=[
            pl.BlockSpec((blk, f), lambda i: (i, 0)),
            pl.BlockSpec((f, h), lambda i: (0, 0)),
        ],
        out_specs=pl.BlockSpec((blk, h), lambda i: (i, 0)),
        out_shape=jax.ShapeDtypeStruct((n, h), jnp.float32),
    )


@functools.lru_cache(maxsize=None)
def _make_scale1(n, h, blk):
    def body(xw_ref, dp_ref, y_ref):
        y_ref[...] = _dinv_from(dp_ref) * xw_ref[...]

    return pl.pallas_call(
        body,
        grid=(n // blk,),
        in_specs=[
            pl.BlockSpec((blk, h), lambda i: (i, 0)),
            pl.BlockSpec((NC, blk, DW), lambda i: (0, i, 0)),
        ],
        out_specs=pl.BlockSpec((blk, h), lambda i: (i, 0)),
        out_shape=jax.ShapeDtypeStruct((n, h), jnp.float32),
    )


@functools.lru_cache(maxsize=None)
def _make_mid(n, h, blk):
    def body(p_ref, dp_ref, w2_ref, b1_ref, y2_ref):
        dinv = _dinv_from(dp_ref)
        agg = p_ref[0] + p_ref[1]
        hact = jnp.maximum(dinv * agg + b1_ref[...], 0.0)
        y2_ref[...] = dinv * jnp.dot(
            hact, w2_ref[...], preferred_element_type=jnp.float32)

    return pl.pallas_call(
        body,
        grid=(n // blk,),
        in_specs=[
            pl.BlockSpec((NC, blk, h), lambda i: (0, i, 0)),
            pl.BlockSpec((NC, blk, DW), lambda i: (0, i, 0)),
            pl.BlockSpec((h, h), lambda i: (0, 0)),
            pl.BlockSpec((1, h), lambda i: (0, 0)),
        ],
        out_specs=pl.BlockSpec((blk, h), lambda i: (i, 0)),
        out_shape=jax.ShapeDtypeStruct((n, h), jnp.float32),
    )


@functools.lru_cache(maxsize=None)
def _make_emb_out(n, h, s_sub, blk):
    def body(p_ref, dp_ref, b2_ref, wl_ref, bl_ref, emb_ref, o_ref):
        dinv = _dinv_from(dp_ref)
        agg = p_ref[0] + p_ref[1]
        emb = dinv * agg + b2_ref[...]
        emb_ref[...] = emb
        z = emb.reshape(blk // s_sub, s_sub * h)
        o_ref[...] = lax.dot_general(
            z, wl_ref[...], (((1,), (1,)), ((), ())),
            preferred_element_type=jnp.float32) + bl_ref[...]

    return pl.pallas_call(
        body,
        grid=(n // blk,),
        in_specs=[
            pl.BlockSpec((NC, blk, h), lambda i: (0, i, 0)),
            pl.BlockSpec((NC, blk, DW), lambda i: (0, i, 0)),
            pl.BlockSpec((1, h), lambda i: (0, 0)),
            pl.BlockSpec((h, s_sub * h), lambda i: (0, 0)),
            pl.BlockSpec((1, h), lambda i: (0, 0)),
        ],
        out_specs=[
            pl.BlockSpec((blk, h), lambda i: (i, 0)),
            pl.BlockSpec((blk // s_sub, h), lambda i: (i, 0)),
        ],
        out_shape=[
            jax.ShapeDtypeStruct((n, h), jnp.float32),
            jax.ShapeDtypeStruct((n // s_sub, h), jnp.float32),
        ],
    )


def kernel(x, edge_index, W1, b1, W2, b2, Wl, bl):
    n, f = x.shape
    h = W1.shape[1]
    e = edge_index.shape[1]
    s_sub = Wl.shape[1] // h

    degp = _make_deg(n, e)(edge_index)
    xw1 = _make_mm1(n, f, h, 2000)(x, W1)
    y1 = _make_scale1(n, h, 2000)(xw1, degp)
    p1 = _make_agg(n, h, e)(y1, edge_index)
    y2 = _make_mid(n, h, 2000)(p1, degp, W2, b1.reshape(1, h))
    p2 = _make_agg(n, h, e)(y2, edge_index)
    emb, out = _make_emb_out(n, h, s_sub, 2000)(
        p2, degp, b2.reshape(1, h), Wl, bl.reshape(1, h))
    return (out, emb)
